# baseline (speedup 1.0000x reference)
import functools

import jax
import jax.numpy as jnp
from jax import lax
from jax.experimental import pallas as pl
from jax.experimental.pallas import tpu as pltpu
from jax.experimental.pallas import tpu_sc as plsc

N = 10000
NPAD = 10240
E = 320000
NS = 16
CH = 40
EPT = E // NS
CPT = EPT // CH
RPT = NPAD // NS
NCH = RPT // CH
ETA = 1e-6


def _sc_edge_pass(edges, xl_s, xr_s, xl_p, xr_p, att_s, att_p, D):
    J = D // 16
    ACCW = D + 16
    mesh = plsc.VectorSubcoreMesh(core_axis_name="c", subcore_axis_name="s")

    @functools.partial(
        pl.kernel,
        mesh=mesh,
        compiler_params=pltpu.CompilerParams(use_tc_tiling_on_sc=False),
        out_type=(jax.ShapeDtypeStruct((NPAD, ACCW), jnp.float32),
                  jax.ShapeDtypeStruct((NPAD, ACCW), jnp.float32)),
        scratch_types=[
            pltpu.VMEM((CH,), jnp.int32),
            pltpu.VMEM((CH,), jnp.int32),
            pltpu.VMEM((CH,), jnp.int32),
            pltpu.VMEM((CH,), jnp.int32),
            pltpu.VMEM((CH, D), jnp.float32),
            pltpu.VMEM((CH, D), jnp.float32),
            pltpu.VMEM((CH, D), jnp.float32),
            pltpu.VMEM((CH, D), jnp.float32),
            pltpu.VMEM((CH, ACCW), jnp.float32),
            pltpu.VMEM((CH, ACCW), jnp.float32),
            pltpu.VMEM((J, 16), jnp.float32),
            pltpu.VMEM_SHARED((NPAD, ACCW), jnp.float32),
        ] + [pltpu.SemaphoreType.DMA] * 4,
    )
    def k(edges_h, xls_h, xrs_h, xlp_h, xrp_h, att_s_h, att_p_h,
          out_s, out_p, src0, dst0, src1, dst1,
          xlb0, xrb0, xlb1, xrb1, row0, row1, att_v,
          acc, s_i0, s_i1, s_g0, s_g1):
        B0 = (src0, dst0, None, xlb0, xrb0, row0, s_i0, s_g0)
        B1 = (src1, dst1, None, xlb1, xrb1, row1, s_i1, s_g1)
        c = lax.axis_index("c")
        s = lax.axis_index("s")
        lane = lax.iota(jnp.int32, 16)
        onehot0 = jnp.maximum(1.0 - lane.astype(jnp.float32), 0.0)
        perms = [lane ^ sh for sh in (1, 2, 4, 8)]
        rb = s * RPT
        ebase = s * EPT

        def run(xl_h, xr_h, att_h, out_h):
            pltpu.sync_copy(att_h, att_v)
            att = [att_v[j] for j in range(J)]

            def edge_rows(i, xlr, xrr, row):
                als = []
                vals = []
                for j in range(J):
                    a = xlr[i, pl.ds(16 * j, 16)]
                    b = xrr[i, pl.ds(16 * j, 16)]
                    m = a + b
                    lr = jnp.maximum(m, 0.2 * m)
                    vals.append(att[j] * lr)
                    als.append(a)
                while len(vals) > 1:
                    vals = ([vals[t] + vals[t + 1]
                             for t in range(0, len(vals) - 1, 2)]
                            + ([vals[-1]] if len(vals) % 2 else []))
                tot = vals[0]
                for p in perms:
                    tot = tot + tot.at[p].get(mode='promise_in_bounds')
                wv = jnp.exp(tot)
                for j in range(J):
                    row[i, pl.ds(16 * j, 16)] = als[j] * wv
                row[i, pl.ds(D, 16)] = wv * onehot0

            @pl.loop(0, NCH)
            def _(b):
                rows = pl.ds(rb + b * CH, CH)
                cp1 = pltpu.async_copy(xl_h.at[rows], xlb0, s_g0)
                cp2 = pltpu.async_copy(xr_h.at[rows], xrb0, s_g1)
                cp1.wait()
                cp2.wait()

                @pl.loop(0, CH)
                def _(i):
                    edge_rows(i, xlb0, xrb0, row0)

                pltpu.sync_copy(row0, acc.at[rows])

            plsc.subcore_barrier()

            def idx_start(B, g):
                base = ebase + g * CH
                pltpu.async_copy(edges_h.at[0, pl.ds(base, CH)], B[0], B[6])
                pltpu.async_copy(edges_h.at[1, pl.ds(base, CH)], B[1], B[6])

            def idx_wait(B):
                pltpu.make_async_copy(
                    edges_h.at[0, pl.ds(0, CH)], B[0], B[6]).wait()
                pltpu.make_async_copy(
                    edges_h.at[1, pl.ds(0, CH)], B[1], B[6]).wait()

            def gather_start(B):
                pltpu.async_copy(xl_h.at[B[0]], B[3], B[7])
                pltpu.async_copy(xr_h.at[B[1]], B[4], B[7])

            def gather_wait(B):
                pltpu.make_async_copy(xl_h.at[B[0]], B[3], B[7]).wait()
                pltpu.make_async_copy(xr_h.at[B[1]], B[4], B[7]).wait()

            def step(g, B, Bn):
                gather_wait(B)

                @pl.when(g + 1 < CPT)
                def _():
                    idx_wait(Bn)
                    gather_start(Bn)

                @pl.loop(0, CH, unroll=4)
                def _(i):
                    edge_rows(i, B[3], B[4], B[5])

                pltpu.sync_copy(B[5], acc.at[B[1]], add=True)

                @pl.when(g + 2 < CPT)
                def _():
                    idx_start(B, g + 2)

            idx_start(B0, 0)
            idx_wait(B0)
            gather_start(B0)
            idx_start(B1, 1)

            @pl.loop(0, CPT, step=2)
            def _(g):
                step(g, B0, B1)
                step(g + 1, B1, B0)

            plsc.subcore_barrier()
            for b in range(NCH):
                sl = pl.ds(rb + b * CH, CH)
                pltpu.sync_copy(acc.at[sl], out_h.at[sl])

        @pl.when(c == 0)
        def _():
            run(xls_h, xrs_h, att_s_h, out_s)

        @pl.when(c == 1)
        def _():
            run(xlp_h, xrp_h, att_p_h, out_p)

    return k(edges, xl_s, xr_s, xl_p, xr_p, att_s, att_p)


_BLK = 512
_GRID = NPAD // _BLK


def _mm(a, b):
    return lax.dot_general(a, b, (((1,), (0,)), ((), ())),
                           precision=lax.Precision.HIGHEST,
                           preferred_element_type=jnp.float32)


def _tc_proj1(x, w1, w2, w3, w4):
    def body(x_ref, w1_ref, w2_ref, w3_ref, w4_ref, o1, o2, o3, o4):
        xb = x_ref[...]
        o1[...] = _mm(xb, w1_ref[...])
        o2[...] = _mm(xb, w2_ref[...])
        o3[...] = _mm(xb, w3_ref[...])
        o4[...] = _mm(xb, w4_ref[...])

    wspec = pl.BlockSpec((128, 128), lambda i: (0, 0))
    ospec = pl.BlockSpec((_BLK, 128), lambda i: (i, 0))
    return pl.pallas_call(
        body,
        grid=(_GRID,),
        in_specs=[pl.BlockSpec((_BLK, 128), lambda i: (i, 0)),
                  wspec, wspec, wspec, wspec],
        out_specs=[ospec, ospec, ospec, ospec],
        out_shape=[jax.ShapeDtypeStruct((NPAD, 128), jnp.float32)] * 4,
    )(x, w1, w2, w3, w4)


def _tc_mid(acc_s, acc_p, bs1, bp1, wsl, wsr, wpl, wpr):
    def body(as_ref, ap_ref, bs_ref, bp_ref, wsl_ref, wsr_ref, wpl_ref,
             wpr_ref, o1, o2, o3, o4):
        a_s = as_ref[...]
        a_p = ap_ref[...]
        s1 = a_s[:, :128] / (a_s[:, 128:129] + 1e-16) + bs_ref[...]
        s1 = jnp.maximum(s1, 0.0)
        p1 = a_p[:, :128] / (a_p[:, 128:129] + 1e-16) + bp_ref[...]
        o1[...] = _mm(s1, wsl_ref[...])
        o2[...] = _mm(s1, wsr_ref[...])
        o3[...] = _mm(p1, wpl_ref[...])
        o4[...] = _mm(p1, wpr_ref[...])

    aspec = pl.BlockSpec((_BLK, 144), lambda i: (i, 0))
    bspec = pl.BlockSpec((1, 128), lambda i: (0, 0))
    wspec = pl.BlockSpec((128, 32), lambda i: (0, 0))
    ospec = pl.BlockSpec((_BLK, 32), lambda i: (i, 0))
    return pl.pallas_call(
        body,
        grid=(_GRID,),
        in_specs=[aspec, aspec, bspec, bspec, wspec, wspec, wspec, wspec],
        out_specs=[ospec, ospec, ospec, ospec],
        out_shape=[jax.ShapeDtypeStruct((NPAD, 32), jnp.float32)] * 4,
    )(acc_s, acc_p, bs1, bp1, wsl, wsr, wpl, wpr)


def _tc_fin(acc_s, acc_p, bs2, bp2):
    def body(as_ref, ap_ref, bs_ref, bp_ref, mu_ref, std_ref):
        a_s = as_ref[...]
        a_p = ap_ref[...]
        sv = a_s[:, :32] / (a_s[:, 32:33] + 1e-16) + bs_ref[...]
        pv = a_p[:, :32] / (a_p[:, 32:33] + 1e-16) + bp_ref[...]
        mu_ref[...] = jnp.concatenate([sv[:, :16], pv[:, :16]], axis=1)
        raw = jnp.concatenate([sv[:, 16:], pv[:, 16:]], axis=1)
        std_ref[...] = jax.nn.softplus(raw) + ETA

    aspec = pl.BlockSpec((_BLK, 48), lambda i: (i, 0))
    bspec = pl.BlockSpec((1, 32), lambda i: (0, 0))
    ospec = pl.BlockSpec((_BLK, 32), lambda i: (i, 0))
    return pl.pallas_call(
        body,
        grid=(_GRID,),
        in_specs=[aspec, aspec, bspec, bspec],
        out_specs=[ospec, ospec],
        out_shape=[jax.ShapeDtypeStruct((NPAD, 32), jnp.float32)] * 2,
    )(acc_s, acc_p, bs2, bp2)


def kernel(x, edge_index, Ws1_l, Ws1_r, atts1, bs1, Ws2_l, Ws2_r, atts2, bs2,
           Wp1_l, Wp1_r, attp1, bp1, Wp2_l, Wp2_r, attp2, bp2):
    xpad = jnp.pad(x, ((0, NPAD - N), (0, 0)))

    ats1 = atts1.reshape(8, 16)
    atp1 = attp1.reshape(8, 16)
    ats2 = atts2.reshape(2, 16)
    atp2 = attp2.reshape(2, 16)
    (edge_index, ats1, atp1, ats2, atp2) = lax.optimization_barrier(
        (edge_index, ats1, atp1, ats2, atp2))

    xls, xrs, xlp, xrp = _tc_proj1(xpad, Ws1_l, Ws1_r, Wp1_l, Wp1_r)
    acc_s, acc_p = _sc_edge_pass(edge_index, xls, xrs, xlp, xrp,
                                 ats1, atp1, 128)
    xl2s, xr2s, xl2p, xr2p = _tc_mid(acc_s, acc_p, bs1.reshape(1, 128),
                                     bp1.reshape(1, 128), Ws2_l, Ws2_r,
                                     Wp2_l, Wp2_r)
    acc2_s, acc2_p = _sc_edge_pass(edge_index, xl2s, xr2s, xl2p, xr2p,
                                   ats2, atp2, 32)
    mu, std = _tc_fin(acc2_s, acc2_p, bs2.reshape(1, 32), bp2.reshape(1, 32))
    return mu[:N], std[:N]


# speedup vs baseline: 1.3534x; 1.3534x over previous
import functools

import jax
import jax.numpy as jnp
from jax import lax
from jax.experimental import pallas as pl
from jax.experimental.pallas import tpu as pltpu
from jax.experimental.pallas import tpu_sc as plsc

N = 10000
NPAD = 10240
E = 320000
NS = 16
CH = 32
EPT = E // NS
CPT = EPT // CH
RPT = NPAD // NS
NCH = RPT // CH
ETA = 1e-6


def _sc_edge_pass(edges, xl_s, xr_s, xl_p, xr_p, att_s, att_p, D):
    J = D // 16
    ACCW = D + 16
    mesh = plsc.VectorSubcoreMesh(core_axis_name="c", subcore_axis_name="s")

    @functools.partial(
        pl.kernel,
        mesh=mesh,
        compiler_params=pltpu.CompilerParams(use_tc_tiling_on_sc=False),
        out_type=(jax.ShapeDtypeStruct((NPAD, ACCW), jnp.float32),
                  jax.ShapeDtypeStruct((NPAD, ACCW), jnp.float32)),
        scratch_types=[
            pltpu.VMEM((2, CH), jnp.int32),
            pltpu.VMEM((CH,), jnp.int32),
            pltpu.VMEM((2, CH), jnp.int32),
            pltpu.VMEM((CH,), jnp.int32),
            pltpu.VMEM((CH, D), jnp.float32),
            pltpu.VMEM((CH, D), jnp.float32),
            pltpu.VMEM((CH, D), jnp.float32),
            pltpu.VMEM((CH, D), jnp.float32),
            pltpu.VMEM((CH, ACCW), jnp.float32),
            pltpu.VMEM((CH, ACCW), jnp.float32),
            pltpu.VMEM((J, 16), jnp.float32),
            pltpu.VMEM_SHARED((NPAD, ACCW), jnp.float32),
        ] + [pltpu.SemaphoreType.DMA] * 6,
    )
    def k(edges_h, xls_h, xrs_h, xlp_h, xrp_h, att_s_h, att_p_h,
          out_s, out_p, idx0, dsc0, idx1, dsc1,
          xlb0, xrb0, xlb1, xrb1, row0, row1, att_v,
          acc, s_i0, s_i1, s_g0, s_g1, s_s0, s_s1):
        B0 = (idx0, dsc0, xlb0, xrb0, row0, s_i0, s_g0, s_s0)
        B1 = (idx1, dsc1, xlb1, xrb1, row1, s_i1, s_g1, s_s1)
        c = lax.axis_index("c")
        s = lax.axis_index("s")
        lane = lax.iota(jnp.int32, 16)
        onehot0 = jnp.maximum(1.0 - lane.astype(jnp.float32), 0.0)
        perms = [lane ^ sh for sh in (1, 2, 4, 8)]
        rb = s * RPT
        ebase = s * EPT

        def run(xl_h, xr_h, att_h, out_h):
            pltpu.sync_copy(att_h, att_v)
            att = [att_v[j] for j in range(J)]

            def edge_rows(i, xlr, xrr, row):
                als = []
                vals = []
                for j in range(J):
                    a = xlr[i, pl.ds(16 * j, 16)]
                    b = xrr[i, pl.ds(16 * j, 16)]
                    m = a + b
                    lr = jnp.maximum(m, 0.2 * m)
                    vals.append(att[j] * lr)
                    als.append(a)
                while len(vals) > 1:
                    vals = ([vals[t] + vals[t + 1]
                             for t in range(0, len(vals) - 1, 2)]
                            + ([vals[-1]] if len(vals) % 2 else []))
                tot = vals[0]
                for p in perms:
                    tot = tot + tot.at[p].get(mode='promise_in_bounds')
                wv = jnp.exp(tot)
                for j in range(J):
                    row[i, pl.ds(16 * j, 16)] = als[j] * wv
                row[i, pl.ds(D, 16)] = wv * onehot0

            @pl.loop(0, NCH)
            def _(b):
                rows = pl.ds(rb + b * CH, CH)
                cp1 = pltpu.async_copy(xl_h.at[rows], xlb0, s_g0)
                cp2 = pltpu.async_copy(xr_h.at[rows], xrb0, s_g1)
                cp1.wait()
                cp2.wait()

                @pl.loop(0, CH)
                def _(i):
                    edge_rows(i, xlb0, xrb0, row0)

                pltpu.sync_copy(row0, acc.at[rows])

            plsc.subcore_barrier()

            def idx_start(B, g):
                base = ebase + g * CH
                pltpu.async_copy(edges_h.at[:, pl.ds(base, CH)], B[0], B[5])

            def idx_wait(B):
                pltpu.make_async_copy(
                    edges_h.at[:, pl.ds(0, CH)], B[0], B[5]).wait()

            def gather_start(B):
                pltpu.async_copy(xl_h.at[B[0].at[0]], B[2], B[6])
                pltpu.async_copy(xr_h.at[B[0].at[1]], B[3], B[6])

            def gather_wait(B):
                pltpu.make_async_copy(xl_h.at[B[0].at[0]], B[2], B[6]).wait()
                pltpu.make_async_copy(xr_h.at[B[0].at[1]], B[3], B[6]).wait()

            def scatter_drain(B):
                pltpu.make_async_copy(out_h.at[pl.ds(0, CH)], B[4],
                                      B[7]).wait()

            def step(g, B, Bn):
                gather_wait(B)

                @pl.when(g >= 2)
                def _():
                    scatter_drain(B)

                for t in range(CH // 16):
                    sl = pl.ds(16 * t, 16)
                    B[1][sl] = B[0][1, sl]

                @pl.when(g + 2 < CPT)
                def _():
                    idx_start(B, g + 2)

                @pl.when(g + 1 < CPT)
                def _():
                    idx_wait(Bn)
                    gather_start(Bn)

                @pl.loop(0, CH, unroll=4)
                def _(i):
                    edge_rows(i, B[2], B[3], B[4])

                pltpu.async_copy(B[4], acc.at[B[1]], B[7], add=True)

            idx_start(B0, 0)
            idx_wait(B0)
            gather_start(B0)
            idx_start(B1, 1)

            @pl.loop(0, CPT - 1, step=2)
            def _(g):
                step(g, B0, B1)
                step(g + 1, B1, B0)

            gather_wait(B0)
            scatter_drain(B0)
            for t in range(CH // 16):
                sl = pl.ds(16 * t, 16)
                B0[1][sl] = B0[0][1, sl]

            @pl.loop(0, CH, unroll=4)
            def _(i):
                edge_rows(i, B0[2], B0[3], B0[4])

            pltpu.async_copy(B0[4], acc.at[B0[1]], B0[7], add=True)

            scatter_drain(B0)
            scatter_drain(B1)
            plsc.subcore_barrier()
            for b in range(NCH):
                sl = pl.ds(rb + b * CH, CH)
                pltpu.sync_copy(acc.at[sl], out_h.at[sl])

        @pl.when(c == 0)
        def _():
            run(xls_h, xrs_h, att_s_h, out_s)

        @pl.when(c == 1)
        def _():
            run(xlp_h, xrp_h, att_p_h, out_p)

    return k(edges, xl_s, xr_s, xl_p, xr_p, att_s, att_p)


_BLK = 512
_GRID = NPAD // _BLK


def _mm(a, b):
    return lax.dot_general(a, b, (((1,), (0,)), ((), ())),
                           precision=lax.Precision.HIGHEST,
                           preferred_element_type=jnp.float32)


def _tc_proj1(x, w1, w2, w3, w4):
    def body(x_ref, w1_ref, w2_ref, w3_ref, w4_ref, o1, o2, o3, o4):
        xb = x_ref[...]
        o1[...] = _mm(xb, w1_ref[...])
        o2[...] = _mm(xb, w2_ref[...])
        o3[...] = _mm(xb, w3_ref[...])
        o4[...] = _mm(xb, w4_ref[...])

    wspec = pl.BlockSpec((128, 128), lambda i: (0, 0))
    ospec = pl.BlockSpec((_BLK, 128), lambda i: (i, 0))
    return pl.pallas_call(
        body,
        grid=(_GRID,),
        in_specs=[pl.BlockSpec((_BLK, 128), lambda i: (i, 0)),
                  wspec, wspec, wspec, wspec],
        out_specs=[ospec, ospec, ospec, ospec],
        out_shape=[jax.ShapeDtypeStruct((NPAD, 128), jnp.float32)] * 4,
    )(x, w1, w2, w3, w4)


def _tc_mid(acc_s, acc_p, bs1, bp1, wsl, wsr, wpl, wpr):
    def body(as_ref, ap_ref, bs_ref, bp_ref, wsl_ref, wsr_ref, wpl_ref,
             wpr_ref, o1, o2, o3, o4):
        a_s = as_ref[...]
        a_p = ap_ref[...]
        s1 = a_s[:, :128] / (a_s[:, 128:129] + 1e-16) + bs_ref[...]
        s1 = jnp.maximum(s1, 0.0)
        p1 = a_p[:, :128] / (a_p[:, 128:129] + 1e-16) + bp_ref[...]
        o1[...] = _mm(s1, wsl_ref[...])
        o2[...] = _mm(s1, wsr_ref[...])
        o3[...] = _mm(p1, wpl_ref[...])
        o4[...] = _mm(p1, wpr_ref[...])

    aspec = pl.BlockSpec((_BLK, 144), lambda i: (i, 0))
    bspec = pl.BlockSpec((1, 128), lambda i: (0, 0))
    wspec = pl.BlockSpec((128, 32), lambda i: (0, 0))
    ospec = pl.BlockSpec((_BLK, 32), lambda i: (i, 0))
    return pl.pallas_call(
        body,
        grid=(_GRID,),
        in_specs=[aspec, aspec, bspec, bspec, wspec, wspec, wspec, wspec],
        out_specs=[ospec, ospec, ospec, ospec],
        out_shape=[jax.ShapeDtypeStruct((NPAD, 32), jnp.float32)] * 4,
    )(acc_s, acc_p, bs1, bp1, wsl, wsr, wpl, wpr)


def _tc_fin(acc_s, acc_p, bs2, bp2):
    def body(as_ref, ap_ref, bs_ref, bp_ref, mu_ref, std_ref):
        a_s = as_ref[...]
        a_p = ap_ref[...]
        sv = a_s[:, :32] / (a_s[:, 32:33] + 1e-16) + bs_ref[...]
        pv = a_p[:, :32] / (a_p[:, 32:33] + 1e-16) + bp_ref[...]
        mu_ref[...] = jnp.concatenate([sv[:, :16], pv[:, :16]], axis=1)
        raw = jnp.concatenate([sv[:, 16:], pv[:, 16:]], axis=1)
        std_ref[...] = jax.nn.softplus(raw) + ETA

    aspec = pl.BlockSpec((_BLK, 48), lambda i: (i, 0))
    bspec = pl.BlockSpec((1, 32), lambda i: (0, 0))
    ospec = pl.BlockSpec((_BLK, 32), lambda i: (i, 0))
    return pl.pallas_call(
        body,
        grid=(_GRID,),
        in_specs=[aspec, aspec, bspec, bspec],
        out_specs=[ospec, ospec],
        out_shape=[jax.ShapeDtypeStruct((NPAD, 32), jnp.float32)] * 2,
    )(acc_s, acc_p, bs2, bp2)


def kernel(x, edge_index, Ws1_l, Ws1_r, atts1, bs1, Ws2_l, Ws2_r, atts2, bs2,
           Wp1_l, Wp1_r, attp1, bp1, Wp2_l, Wp2_r, attp2, bp2):
    xpad = jnp.pad(x, ((0, NPAD - N), (0, 0)))

    ats1 = atts1.reshape(8, 16)
    atp1 = attp1.reshape(8, 16)
    ats2 = atts2.reshape(2, 16)
    atp2 = attp2.reshape(2, 16)
    (edge_index, ats1, atp1, ats2, atp2) = lax.optimization_barrier(
        (edge_index, ats1, atp1, ats2, atp2))

    xls, xrs, xlp, xrp = _tc_proj1(xpad, Ws1_l, Ws1_r, Wp1_l, Wp1_r)
    acc_s, acc_p = _sc_edge_pass(edge_index, xls, xrs, xlp, xrp,
                                 ats1, atp1, 128)
    xl2s, xr2s, xl2p, xr2p = _tc_mid(acc_s, acc_p, bs1.reshape(1, 128),
                                     bp1.reshape(1, 128), Ws2_l, Ws2_r,
                                     Wp2_l, Wp2_r)
    acc2_s, acc2_p = _sc_edge_pass(edge_index, xl2s, xr2s, xl2p, xr2p,
                                   ats2, atp2, 32)
    mu, std = _tc_fin(acc2_s, acc2_p, bs2.reshape(1, 32), bp2.reshape(1, 32))
    return mu[:N], std[:N]
